# Initial kernel scaffold; baseline (speedup 1.0000x reference)
import jax
import jax.numpy as jnp
from jax.experimental import pallas as pl


def kernel(x, edge_index, W1, b1, W2, b2):
    raise NotImplementedError("write your pallas kernel here")


# speedup vs baseline: 8.3665x; 8.3665x over previous
import functools

import jax
import jax.numpy as jnp
from jax import lax
from jax.experimental import pallas as pl
from jax.experimental.pallas import tpu as pltpu
from jax.experimental.pallas import tpu_sc as plsc

N = 10000
NP = 10240
E = 320000
B = 128
NC = 2
NS = 16
EP = 327680
NBB = EP // B
RT = NP // NS
RB = 1024


def _fill_rows(ref, rows, cols, value):
    v16 = jnp.full((16,), value, jnp.float32)
    per_row = cols // 16

    def body(i, _):
        r = i // per_row
        j = i % per_row
        ref[r, pl.ds(j * 16, 16)] = v16
        return 0

    lax.fori_loop(0, rows * per_row, body, 0)


def _make_sc_pass(F, nbt, edge_split):
    zcopies = RT // B
    slab = 16
    nslab = nbt // slab

    scratch = [
        pltpu.VMEM((slab, B), jnp.int32),
        pltpu.VMEM((slab, B), jnp.int32),
        pltpu.VMEM((B, F), jnp.float32),
        pltpu.VMEM_SHARED((NP, F), jnp.float32),
        pltpu.SemaphoreType.DMA,
    ]

    def body(tab, srcs, dsts, out, idxs_v, idxd_v, rows_v, acc_sh, sem):
        c = lax.axis_index("c")
        s = lax.axis_index("s")
        if edge_split:
            boff = c * (nbt * NS) + s * nbt
        else:
            boff = s * nbt

        _fill_rows(rows_v, B, F, 0.0)
        for k in range(zcopies):
            pltpu.sync_copy(rows_v, acc_sh.at[pl.ds(s * RT + k * B, B)])

        plsc.subcore_barrier()

        def slab_body(si, carry):
            sboff = boff + si * slab
            pltpu.sync_copy(dsts.at[pl.ds(sboff, slab)], idxd_v)
            if edge_split:
                pltpu.sync_copy(srcs.at[pl.ds(sboff, slab)], idxs_v)
            else:
                pltpu.sync_copy(srcs.at[c, pl.ds(sboff, slab)], idxs_v)

            def step(b, inner):
                pltpu.async_copy(tab.at[idxs_v.at[b]], rows_v, sem).wait()
                pltpu.sync_copy(rows_v, acc_sh.at[idxd_v.at[b]], add=True)
                return inner

            lax.fori_loop(0, slab, step, 0)
            return carry

        lax.fori_loop(0, nslab, slab_body, 0)

        plsc.subcore_barrier()
        pltpu.sync_copy(acc_sh.at[pl.ds(s * RT, RT)],
                        out.at[c, pl.ds(s * RT, RT)])

    return pl.kernel(
        body,
        out_type=jax.ShapeDtypeStruct((NC, NP, F), jnp.float32),
        mesh=plsc.VectorSubcoreMesh(core_axis_name="c", subcore_axis_name="s"),
        scratch_types=scratch,
    )


def _make_deg_pass():
    F = 16
    nbt = NBB // (NC * NS)
    slab = 16
    nslab = nbt // slab
    zcopies = RT // B

    def body(ones_hbm, dsts, out, idxd_v, rows_v, acc_sh):
        c = lax.axis_index("c")
        s = lax.axis_index("s")
        boff = c * (nbt * NS) + s * nbt
        _fill_rows(rows_v, B, F, 0.0)
        for k in range(zcopies):
            pltpu.sync_copy(rows_v, acc_sh.at[pl.ds(s * RT + k * B, B)])
        pltpu.sync_copy(ones_hbm, rows_v)
        plsc.subcore_barrier()

        def slab_body(si, carry):
            pltpu.sync_copy(dsts.at[pl.ds(boff + si * slab, slab)], idxd_v)

            def step(b, inner):
                pltpu.sync_copy(rows_v, acc_sh.at[idxd_v.at[b]], add=True)
                return inner
            lax.fori_loop(0, slab, step, 0)
            return carry
        lax.fori_loop(0, nslab, slab_body, 0)

        plsc.subcore_barrier()
        pltpu.sync_copy(acc_sh.at[pl.ds(s * RT, RT)],
                        out.at[c, pl.ds(s * RT, RT)])

    return pl.kernel(
        body,
        out_type=jax.ShapeDtypeStruct((NC, NP, F), jnp.float32),
        mesh=plsc.VectorSubcoreMesh(core_axis_name="c", subcore_axis_name="s"),
        scratch_types=[
            pltpu.VMEM((slab, B), jnp.int32),
            pltpu.VMEM((B, F), jnp.float32),
            pltpu.VMEM_SHARED((NP, F), jnp.float32),
        ],
        compiler_params=pltpu.CompilerParams(use_tc_tiling_on_sc=False),
    )


def _dinv_of(d_ref):
    deg = d_ref[0] + d_ref[1] + 1.0
    return lax.rsqrt(jnp.maximum(deg, 1.0))


def _tc1_body(x_ref, w_ref, d_ref, o_ref):
    dinv = _dinv_of(d_ref)
    h = jnp.dot(x_ref[...], w_ref[...], preferred_element_type=jnp.float32)
    o_ref[0] = h * dinv[:, None]


def _tc2_body(a_ref, g_ref, d_ref, b_ref, w_ref, o_ref):
    dinv = _dinv_of(d_ref)
    h2 = None
    for cc in range(2):
        r = (a_ref[cc] + g_ref[cc]) * dinv[:, None] + b_ref[cc][None, :]
        r = jnp.maximum(r, 0.0)
        p = jnp.dot(r, w_ref[cc], preferred_element_type=jnp.float32)
        h2 = p if h2 is None else h2 + p
    o_ref[...] = h2 * dinv[:, None]


def _tc3_body(a_ref, g_ref, d_ref, b_ref, o_ref):
    dinv = _dinv_of(d_ref)
    o_ref[...] = ((a_ref[0] + a_ref[1] + g_ref[...]) * dinv[:, None]
                  + b_ref[0][None, :])


@jax.jit
def kernel(x, edge_index, W1, b1, W2, b2):
    src = edge_index[0]
    dst = edge_index[1]
    padn = EP - E
    fill = jnp.full((padn,), N, jnp.int32)
    src2d = jnp.concatenate([src, fill]).reshape(NBB, B)
    dst2d = jnp.concatenate([dst, fill]).reshape(NBB, B)
    srcs_l1 = jnp.stack([src2d, src2d + NP])
    xp = jnp.zeros((NP, 128), jnp.float32).at[:N].set(x)
    b1r = b1.reshape(2, 128)
    w2r = W2.reshape(2, 128, 128)
    b2r = b2.reshape(1, 128)

    ones_hbm = jnp.ones((B, 16), jnp.float32)
    degp = _make_deg_pass()(ones_hbm, dst2d)
    degcol = degp[:, :, 0]

    g1c = pl.pallas_call(
        _tc1_body,
        grid=(2, NP // RB),
        in_specs=[
            pl.BlockSpec((RB, 128), lambda c, r: (r, 0)),
            pl.BlockSpec((128, 128), lambda c, r: (0, c)),
            pl.BlockSpec((2, RB), lambda c, r: (0, r)),
        ],
        out_specs=pl.BlockSpec((1, RB, 128), lambda c, r: (c, r, 0)),
        out_shape=jax.ShapeDtypeStruct((2, NP, 128), jnp.float32),
    )(xp, W1, degcol)

    tab1 = g1c.reshape(NC * NP, 128)
    acc1 = _make_sc_pass(128, NBB // NS, False)(tab1, srcs_l1, dst2d)

    g2 = pl.pallas_call(
        _tc2_body,
        grid=(NP // RB,),
        in_specs=[
            pl.BlockSpec((2, RB, 128), lambda r: (0, r, 0)),
            pl.BlockSpec((2, RB, 128), lambda r: (0, r, 0)),
            pl.BlockSpec((2, RB), lambda r: (0, r)),
            pl.BlockSpec((2, 128), lambda r: (0, 0)),
            pl.BlockSpec((2, 128, 128), lambda r: (0, 0, 0)),
        ],
        out_specs=pl.BlockSpec((RB, 128), lambda r: (r, 0)),
        out_shape=jax.ShapeDtypeStruct((NP, 128), jnp.float32),
    )(acc1, g1c, degcol, b1r, w2r)

    acc2 = _make_sc_pass(128, NBB // (NC * NS), True)(g2, src2d, dst2d)

    z = pl.pallas_call(
        _tc3_body,
        grid=(NP // RB,),
        in_specs=[
            pl.BlockSpec((2, RB, 128), lambda r: (0, r, 0)),
            pl.BlockSpec((RB, 128), lambda r: (r, 0)),
            pl.BlockSpec((2, RB), lambda r: (0, r)),
            pl.BlockSpec((1, 128), lambda r: (0, 0)),
        ],
        out_specs=pl.BlockSpec((RB, 128), lambda r: (r, 0)),
        out_shape=jax.ShapeDtypeStruct((NP, 128), jnp.float32),
    )(acc2, g2, degcol, b2r)

    return z[:N]


# speedup vs baseline: 9.2498x; 1.1056x over previous
import functools

import jax
import jax.numpy as jnp
from jax import lax
from jax.experimental import pallas as pl
from jax.experimental.pallas import tpu as pltpu
from jax.experimental.pallas import tpu_sc as plsc

N = 10000
NP = 10240
E = 320000
B = 128
NC = 2
NS = 16
EP = 327680
NBB = EP // B
RT = NP // NS
RB = 1024


def _fill_rows(ref, rows, cols, value):
    v16 = jnp.full((16,), value, jnp.float32)
    per_row = cols // 16

    def body(i, _):
        r = i // per_row
        j = i % per_row
        ref[r, pl.ds(j * 16, 16)] = v16
        return 0

    lax.fori_loop(0, rows * per_row, body, 0)


def _make_sc_pass(F, nbt, edge_split):
    zcopies = RT // B
    slab = 16
    nslab = nbt // slab

    scratch = [
        pltpu.VMEM((slab, B), jnp.int32),
        pltpu.VMEM((slab, B), jnp.int32),
        pltpu.VMEM((B, F), jnp.float32),
        pltpu.VMEM((B, F), jnp.float32),
        pltpu.VMEM_SHARED((NP, F), jnp.float32),
        pltpu.SemaphoreType.DMA,
        pltpu.SemaphoreType.DMA,
        pltpu.SemaphoreType.DMA,
        pltpu.SemaphoreType.DMA,
    ]

    def body(tab, srcs, dsts, out, idxs_v, idxd_v, rows0, rows1,
             acc_sh, gsem0, gsem1, ssem0, ssem1):
        c = lax.axis_index("c")
        s = lax.axis_index("s")
        rows = [rows0, rows1]
        gsem = [gsem0, gsem1]
        ssem = [ssem0, ssem1]
        if edge_split:
            boff = c * (nbt * NS) + s * nbt
        else:
            boff = s * nbt

        _fill_rows(rows0, B, F, 0.0)
        for k in range(zcopies):
            pltpu.sync_copy(rows0, acc_sh.at[pl.ds(s * RT + k * B, B)])

        plsc.subcore_barrier()

        def slab_body(si, carry):
            sboff = boff + si * slab
            pltpu.sync_copy(dsts.at[pl.ds(sboff, slab)], idxd_v)
            if edge_split:
                pltpu.sync_copy(srcs.at[pl.ds(sboff, slab)], idxs_v)
            else:
                pltpu.sync_copy(srcs.at[c, pl.ds(sboff, slab)], idxs_v)

            gd = {0: pltpu.async_copy(tab.at[idxs_v.at[0]], rows[0], gsem[0])}
            sd = {}
            for b in range(slab):
                cur = b % 2
                nxt = (b + 1) % 2
                gd[b].wait()
                if b + 1 < slab:
                    if b - 1 in sd:
                        sd[b - 1].wait()
                    gd[b + 1] = pltpu.async_copy(
                        tab.at[idxs_v.at[b + 1]], rows[nxt], gsem[nxt])
                sd[b] = pltpu.async_copy(
                    rows[cur], acc_sh.at[idxd_v.at[b]], ssem[cur], add=True)
            sd[slab - 1].wait()
            if slab >= 2:
                sd[slab - 2].wait()
            return carry

        lax.fori_loop(0, nslab, slab_body, 0)

        plsc.subcore_barrier()
        pltpu.sync_copy(acc_sh.at[pl.ds(s * RT, RT)],
                        out.at[c, pl.ds(s * RT, RT)])

    return pl.kernel(
        body,
        out_type=jax.ShapeDtypeStruct((NC, NP, F), jnp.float32),
        mesh=plsc.VectorSubcoreMesh(core_axis_name="c", subcore_axis_name="s"),
        scratch_types=scratch,
    )


def _make_deg_pass():
    F = 16
    nbt = NBB // (NC * NS)
    slab = 16
    nslab = nbt // slab
    zcopies = RT // B

    def body(ones_hbm, dsts, out, idxd_v, rows_v, acc_sh):
        c = lax.axis_index("c")
        s = lax.axis_index("s")
        boff = c * (nbt * NS) + s * nbt
        _fill_rows(rows_v, B, F, 0.0)
        for k in range(zcopies):
            pltpu.sync_copy(rows_v, acc_sh.at[pl.ds(s * RT + k * B, B)])
        pltpu.sync_copy(ones_hbm, rows_v)
        plsc.subcore_barrier()

        def slab_body(si, carry):
            pltpu.sync_copy(dsts.at[pl.ds(boff + si * slab, slab)], idxd_v)

            def step(b, inner):
                pltpu.sync_copy(rows_v, acc_sh.at[idxd_v.at[b]], add=True)
                return inner
            lax.fori_loop(0, slab, step, 0)
            return carry
        lax.fori_loop(0, nslab, slab_body, 0)

        plsc.subcore_barrier()
        pltpu.sync_copy(acc_sh.at[pl.ds(s * RT, RT)],
                        out.at[c, pl.ds(s * RT, RT)])

    return pl.kernel(
        body,
        out_type=jax.ShapeDtypeStruct((NC, NP, F), jnp.float32),
        mesh=plsc.VectorSubcoreMesh(core_axis_name="c", subcore_axis_name="s"),
        scratch_types=[
            pltpu.VMEM((slab, B), jnp.int32),
            pltpu.VMEM((B, F), jnp.float32),
            pltpu.VMEM_SHARED((NP, F), jnp.float32),
        ],
        compiler_params=pltpu.CompilerParams(use_tc_tiling_on_sc=False),
    )


def _dinv_of(d_ref):
    deg = d_ref[0] + d_ref[1] + 1.0
    return lax.rsqrt(jnp.maximum(deg, 1.0))


def _tc1_body(x_ref, w_ref, d_ref, o_ref):
    dinv = _dinv_of(d_ref)
    h = jnp.dot(x_ref[...], w_ref[...], preferred_element_type=jnp.float32)
    o_ref[0] = h * dinv[:, None]


def _tc2_body(a_ref, g_ref, d_ref, b_ref, w_ref, o_ref):
    dinv = _dinv_of(d_ref)
    h2 = None
    for cc in range(2):
        r = (a_ref[cc] + g_ref[cc]) * dinv[:, None] + b_ref[cc][None, :]
        r = jnp.maximum(r, 0.0)
        p = jnp.dot(r, w_ref[cc], preferred_element_type=jnp.float32)
        h2 = p if h2 is None else h2 + p
    o_ref[...] = h2 * dinv[:, None]


def _tc3_body(a_ref, g_ref, d_ref, b_ref, o_ref):
    dinv = _dinv_of(d_ref)
    o_ref[...] = ((a_ref[0] + a_ref[1] + g_ref[...]) * dinv[:, None]
                  + b_ref[0][None, :])


@jax.jit
def kernel(x, edge_index, W1, b1, W2, b2):
    src = edge_index[0]
    dst = edge_index[1]
    padn = EP - E
    fill = jnp.full((padn,), N, jnp.int32)
    src2d = jnp.concatenate([src, fill]).reshape(NBB, B)
    dst2d = jnp.concatenate([dst, fill]).reshape(NBB, B)
    srcs_l1 = jnp.stack([src2d, src2d + NP])
    xp = jnp.zeros((NP, 128), jnp.float32).at[:N].set(x)
    b1r = b1.reshape(2, 128)
    w2r = W2.reshape(2, 128, 128)
    b2r = b2.reshape(1, 128)

    ones_hbm = jnp.ones((B, 16), jnp.float32)
    degp = _make_deg_pass()(ones_hbm, dst2d)
    degcol = degp[:, :, 0]

    g1c = pl.pallas_call(
        _tc1_body,
        grid=(2, NP // RB),
        in_specs=[
            pl.BlockSpec((RB, 128), lambda c, r: (r, 0)),
            pl.BlockSpec((128, 128), lambda c, r: (0, c)),
            pl.BlockSpec((2, RB), lambda c, r: (0, r)),
        ],
        out_specs=pl.BlockSpec((1, RB, 128), lambda c, r: (c, r, 0)),
        out_shape=jax.ShapeDtypeStruct((2, NP, 128), jnp.float32),
    )(xp, W1, degcol)

    tab1 = g1c.reshape(NC * NP, 128)
    acc1 = _make_sc_pass(128, NBB // NS, False)(tab1, srcs_l1, dst2d)

    g2 = pl.pallas_call(
        _tc2_body,
        grid=(NP // RB,),
        in_specs=[
            pl.BlockSpec((2, RB, 128), lambda r: (0, r, 0)),
            pl.BlockSpec((2, RB, 128), lambda r: (0, r, 0)),
            pl.BlockSpec((2, RB), lambda r: (0, r)),
            pl.BlockSpec((2, 128), lambda r: (0, 0)),
            pl.BlockSpec((2, 128, 128), lambda r: (0, 0, 0)),
        ],
        out_specs=pl.BlockSpec((RB, 128), lambda r: (r, 0)),
        out_shape=jax.ShapeDtypeStruct((NP, 128), jnp.float32),
    )(acc1, g1c, degcol, b1r, w2r)

    acc2 = _make_sc_pass(128, NBB // (NC * NS), True)(g2, src2d, dst2d)

    z = pl.pallas_call(
        _tc3_body,
        grid=(NP // RB,),
        in_specs=[
            pl.BlockSpec((2, RB, 128), lambda r: (0, r, 0)),
            pl.BlockSpec((RB, 128), lambda r: (r, 0)),
            pl.BlockSpec((2, RB), lambda r: (0, r)),
            pl.BlockSpec((1, 128), lambda r: (0, 0)),
        ],
        out_specs=pl.BlockSpec((RB, 128), lambda r: (r, 0)),
        out_shape=jax.ShapeDtypeStruct((NP, 128), jnp.float32),
    )(acc2, g2, degcol, b2r)

    return z[:N]
